# baseline (speedup 1.0000x reference)
import functools

import jax
import jax.numpy as jnp
from jax import lax
from jax.experimental import pallas as pl
from jax.experimental.pallas import tpu as pltpu
from jax.experimental.pallas import tpu_sc as plsc

E = 8
K = 2
T = 2048
D = 768
P = T * K
MM_TILE = 128
NS = P + E * MM_TILE
NT = NS // MM_TILE
NTP = 48
NC = 2
NSUB = 16
TPS = T // NSUB
PPS = TPS * K
SPW = NS // (NC * NSUB)
GCH = 5
NEG_INF = float("-inf")


def _iota16():
    return lax.broadcasted_iota(jnp.int32, (16,), 0)


def _lane_gather(src, idx):
    return lax.gather(
        src, idx[:, None],
        lax.GatherDimensionNumbers(
            offset_dims=(), collapsed_slice_dims=(0,), start_index_map=(0,)),
        slice_sizes=(1,),
        mode=lax.GatherScatterMode.PROMISE_IN_BOUNDS)


def _lane_sum(x):
    iot = _iota16()
    for d in (1, 2, 4, 8):
        x = x + _lane_gather(x, iot ^ d)
    return x


def _lane_prefix(x):
    iot = _iota16()
    for d in (1, 2, 4, 8):
        sh = _lane_gather(x, jnp.maximum(iot - d, 0))
        x = x + jnp.where(iot >= d, sh, 0)
    return x


def _lane_bcast(x, e):
    return _lane_gather(x, jnp.full((16,), e, jnp.int32))


def _dispatch_body(logits_ref, hidden_ref,
                   xs_ref, ws_ref, pos_ref, texp_ref,
                   lg, epair, wpair, histv, allh, texp_v, slots, toks,
                   idxq, rows, ws_v, shared_hist, shared_src, shared_ws,
                   sem, sem2):
    c = lax.axis_index("c")
    s = lax.axis_index("s")

    pltpu.sync_copy(logits_ref.at[:, pl.ds(TPS * s, TPS)], lg)
    for g in range(TPS // 16):
        l_vecs = [lg[e, pl.ds(16 * g, 16)] for e in range(E)]
        best = l_vecs[0]
        bi = jnp.zeros((16,), jnp.int32)
        for e in range(1, E):
            m = l_vecs[e] > best
            best = jnp.where(m, l_vecs[e], best)
            bi = jnp.where(m, e, bi)
        sec = jnp.full((16,), NEG_INF, jnp.float32)
        si = jnp.zeros((16,), jnp.int32)
        for e in range(E):
            m = jnp.logical_and(bi != e, l_vecs[e] > sec)
            sec = jnp.where(m, l_vecs[e], sec)
            si = jnp.where(m, e, si)
        e2 = jnp.exp(sec - best)
        w1 = 1.0 / (1.0 + e2)
        epair[pl.ds(16 * g, 16)] = bi
        epair[pl.ds(TPS + 16 * g, 16)] = si
        wpair[pl.ds(16 * g, 16)] = w1
        wpair[pl.ds(TPS + 16 * g, 16)] = e2 * w1

    iot = _iota16()
    accs = [jnp.zeros((16,), jnp.int32) for _ in range(E)]
    for ch in range(PPS // 16):
        ev = epair[pl.ds(16 * ch, 16)]
        for e in range(E):
            accs[e] = accs[e] + jnp.where(ev == e, 1, 0)
    hist = jnp.zeros((16,), jnp.int32)
    for e in range(E):
        hist = hist + jnp.where(iot == e, _lane_sum(accs[e]), 0)
    histv[...] = hist
    pltpu.sync_copy(histv, shared_hist.at[s])
    plsc.subcore_barrier()

    pltpu.sync_copy(shared_hist, allh)
    tot = jnp.zeros((16,), jnp.int32)
    pre = jnp.zeros((16,), jnp.int32)
    for w in range(NSUB):
        row = allh[w]
        tot = tot + row
        gate = (jnp.int32(w) < s).astype(jnp.int32)
        pre = pre + row * gate
    padded = ((tot + (MM_TILE - 1)) >> 7) << 7
    incl = _lane_prefix(padded)
    base = (incl - padded) + pre

    @pl.when(jnp.logical_and(c == 0, s == 0))
    def _texp():
        ies = [_lane_bcast(incl, e) for e in range(E)]
        for vi in range(NTP // 16):
            startv = (iot + 16 * vi) * MM_TILE
            below = jnp.zeros((16,), jnp.int32)
            for e in range(E):
                below = below + jnp.where(ies[e] <= startv, 1, 0)
            texp_v[pl.ds(16 * vi, 16)] = jnp.minimum(below, E - 1)
        pltpu.sync_copy(texp_v, texp_ref)

    run = base
    for ch in range(PPS // 16):
        ev = epair[pl.ds(16 * ch, 16)]
        sv = jnp.zeros((16,), jnp.int32)
        cvec = jnp.zeros((16,), jnp.int32)
        for e in range(E):
            m = ev == e
            r = _lane_prefix(jnp.where(m, 1, 0))
            sv = jnp.where(m, _lane_bcast(run, e) + (r - 1), sv)
            cvec = cvec + jnp.where(iot == e, _lane_bcast(r, 15), 0)
        run = run + cvec
        slots[pl.ds(16 * ch, 16)] = sv
        toks[pl.ds(16 * ch, 16)] = ((iot + 16 * ch) & (TPS - 1)) + TPS * s

    @pl.when(c == 0)
    def _pos():
        pltpu.sync_copy(slots.at[pl.ds(0, TPS)], pos_ref.at[0, pl.ds(TPS * s, TPS)])
        pltpu.sync_copy(slots.at[pl.ds(TPS, TPS)], pos_ref.at[1, pl.ds(TPS * s, TPS)])

    pltpu.sync_copy(toks, shared_src.at[slots])
    pltpu.sync_copy(wpair, shared_ws.at[slots])
    plsc.subcore_barrier()

    start = SPW * (NSUB * c + s)
    pltpu.sync_copy(shared_ws.at[pl.ds(start, SPW)], ws_v)
    pltpu.sync_copy(ws_v, ws_ref.at[pl.ds(start, SPW)])
    cw = SPW // GCH
    for h in range(GCH):
        pltpu.sync_copy(shared_src.at[pl.ds(start + cw * h, cw)], idxq.at[h])
        for j in range(cw // 16):
            v = idxq[h, pl.ds(16 * j, 16)]
            idxq[h, pl.ds(16 * j, 16)] = jnp.clip(v, 0, T - 1)
    gathers = [
        pltpu.async_copy(hidden_ref.at[idxq.at[h]],
                         rows.at[pl.ds(cw * h, cw)], sem)
        for h in range(GCH)
    ]
    outs = []
    for h in range(GCH):
        gathers[h].wait()
        outs.append(pltpu.async_copy(
            rows.at[pl.ds(cw * h, cw)],
            xs_ref.at[pl.ds(start + cw * h, cw)], sem2))
    for o in outs:
        o.wait()


_dispatch = functools.partial(
    pl.kernel,
    out_type=[
        jax.ShapeDtypeStruct((NS, D), jnp.float32),
        jax.ShapeDtypeStruct((NS,), jnp.float32),
        jax.ShapeDtypeStruct((K, T), jnp.int32),
        jax.ShapeDtypeStruct((NTP,), jnp.int32),
    ],
    mesh=plsc.VectorSubcoreMesh(core_axis_name="c", subcore_axis_name="s"),
    scratch_types=[
        pltpu.VMEM((E, TPS), jnp.float32),
        pltpu.VMEM((PPS,), jnp.int32),
        pltpu.VMEM((PPS,), jnp.float32),
        pltpu.VMEM((16,), jnp.int32),
        pltpu.VMEM((NSUB, 16), jnp.int32),
        pltpu.VMEM((NTP,), jnp.int32),
        pltpu.VMEM((PPS,), jnp.int32),
        pltpu.VMEM((PPS,), jnp.int32),
        pltpu.VMEM((GCH, SPW // GCH), jnp.int32),
        pltpu.VMEM((SPW, D), jnp.float32),
        pltpu.VMEM((SPW,), jnp.float32),
        pltpu.VMEM_SHARED((NSUB, 16), jnp.int32),
        pltpu.VMEM_SHARED((NS,), jnp.int32),
        pltpu.VMEM_SHARED((NS,), jnp.float32),
        pltpu.SemaphoreType.DMA,
        pltpu.SemaphoreType.DMA,
    ],
)(_dispatch_body)


def _mm_body(texp_ref, x_ref, w_ref, b_ref, ws_ref, y_ref, wb_scr):
    i = pl.program_id(0)
    changed = jnp.logical_or(
        i == 0, texp_ref[i] != texp_ref[jnp.maximum(i - 1, 0)])

    @pl.when(changed)
    def _cast():
        wb_scr[...] = w_ref[0].astype(jnp.bfloat16)

    y = jax.lax.dot_general(
        x_ref[...].astype(jnp.bfloat16), wb_scr[...], (((1,), (0,)), ((), ())),
        preferred_element_type=jnp.float32,
    ) + b_ref[0]
    y_ref[...] = y * ws_ref[0, 0][:, None]


def _combine_body(y_ref, pos_ref, out_ref, p0, p1, y0, y1, sem0, sem1):
    wid = lax.axis_index("s") * NC + lax.axis_index("c")
    tw = T // (NC * NSUB)
    base = tw * wid
    pltpu.sync_copy(pos_ref.at[0, pl.ds(base, tw)], p0)
    pltpu.sync_copy(pos_ref.at[1, pl.ds(base, tw)], p1)
    cp0 = pltpu.async_copy(y_ref.at[p0], y0, sem0)
    cp1 = pltpu.async_copy(y_ref.at[p1], y1, sem1)
    cp0.wait()
    cp1.wait()

    def body(t, carry):
        for v in range(D // 16):
            sl = pl.ds(16 * v, 16)
            y0[t, sl] = y0[t, sl] + y1[t, sl]
        return carry

    lax.fori_loop(0, tw, body, 0)
    pltpu.sync_copy(y0, out_ref.at[pl.ds(base, tw)])


_combine = functools.partial(
    pl.kernel,
    out_type=jax.ShapeDtypeStruct((T, D), jnp.float32),
    mesh=plsc.VectorSubcoreMesh(core_axis_name="c", subcore_axis_name="s"),
    scratch_types=[
        pltpu.VMEM((T // (NC * NSUB),), jnp.int32),
        pltpu.VMEM((T // (NC * NSUB),), jnp.int32),
        pltpu.VMEM((T // (NC * NSUB), D), jnp.float32),
        pltpu.VMEM((T // (NC * NSUB), D), jnp.float32),
        pltpu.SemaphoreType.DMA,
        pltpu.SemaphoreType.DMA,
    ],
)(_combine_body)


def kernel(hidden, gate_logits, W_experts, b_experts):
    xs, ws, pos, texp = _dispatch(gate_logits.T, hidden)
    y = pl.pallas_call(
        _mm_body,
        grid_spec=pltpu.PrefetchScalarGridSpec(
            num_scalar_prefetch=1,
            grid=(NT,),
            in_specs=[
                pl.BlockSpec((MM_TILE, D), lambda i, tx: (i, 0)),
                pl.BlockSpec((1, D, D), lambda i, tx: (tx[i], 0, 0)),
                pl.BlockSpec((1, 1, D), lambda i, tx: (tx[i], 0, 0)),
                pl.BlockSpec((1, 1, MM_TILE), lambda i, tx: (i, 0, 0)),
            ],
            out_specs=pl.BlockSpec((MM_TILE, D), lambda i, tx: (i, 0)),
            scratch_shapes=[pltpu.VMEM((D, D), jnp.bfloat16)],
        ),
        out_shape=jax.ShapeDtypeStruct((NS, D), jnp.float32),
    )(texp, xs, W_experts, b_experts.reshape(E, 1, D), ws.reshape(NT, 1, MM_TILE))
    return _combine(y, pos)


# speedup vs baseline: 1.0155x; 1.0155x over previous
import functools

import jax
import jax.numpy as jnp
from jax import lax
from jax.experimental import pallas as pl
from jax.experimental.pallas import tpu as pltpu
from jax.experimental.pallas import tpu_sc as plsc

E = 8
K = 2
T = 2048
D = 768
P = T * K
MM_TILE = 128
NS = P + E * MM_TILE
NT = NS // MM_TILE
NTP = 48
NC = 2
NSUB = 16
TPS = T // NSUB
PPS = TPS * K
SPW = NS // (NC * NSUB)
GCH = 5
NEG_INF = float("-inf")


def _iota16():
    return lax.broadcasted_iota(jnp.int32, (16,), 0)


def _lane_gather(src, idx):
    return lax.gather(
        src, idx[:, None],
        lax.GatherDimensionNumbers(
            offset_dims=(), collapsed_slice_dims=(0,), start_index_map=(0,)),
        slice_sizes=(1,),
        mode=lax.GatherScatterMode.PROMISE_IN_BOUNDS)


def _lane_sum(x):
    iot = _iota16()
    for d in (1, 2, 4, 8):
        x = x + _lane_gather(x, iot ^ d)
    return x


def _lane_prefix(x):
    iot = _iota16()
    for d in (1, 2, 4, 8):
        sh = _lane_gather(x, jnp.maximum(iot - d, 0))
        x = x + jnp.where(iot >= d, sh, 0)
    return x


def _lane_bcast(x, e):
    return _lane_gather(x, jnp.full((16,), e, jnp.int32))


def _dispatch_body(logits_ref, hidden_ref,
                   xs_ref, ws_ref, pos_ref, texp_ref,
                   lg, epair, wpair, histv, allh, texp_v, slots, toks,
                   idxq, rows, ws_v, shared_hist, shared_src, shared_ws,
                   sem, sem2):
    c = lax.axis_index("c")
    s = lax.axis_index("s")

    pltpu.sync_copy(logits_ref.at[:, pl.ds(TPS * s, TPS)], lg)
    for g in range(TPS // 16):
        l_vecs = [lg[e, pl.ds(16 * g, 16)] for e in range(E)]
        best = l_vecs[0]
        bi = jnp.zeros((16,), jnp.int32)
        for e in range(1, E):
            m = l_vecs[e] > best
            best = jnp.where(m, l_vecs[e], best)
            bi = jnp.where(m, e, bi)
        sec = jnp.full((16,), NEG_INF, jnp.float32)
        si = jnp.zeros((16,), jnp.int32)
        for e in range(E):
            m = jnp.logical_and(bi != e, l_vecs[e] > sec)
            sec = jnp.where(m, l_vecs[e], sec)
            si = jnp.where(m, e, si)
        e2 = jnp.exp(sec - best)
        w1 = 1.0 / (1.0 + e2)
        epair[pl.ds(16 * g, 16)] = bi
        epair[pl.ds(TPS + 16 * g, 16)] = si
        wpair[pl.ds(16 * g, 16)] = w1
        wpair[pl.ds(TPS + 16 * g, 16)] = e2 * w1

    iot = _iota16()
    accs = [jnp.zeros((16,), jnp.int32) for _ in range(E)]
    for ch in range(PPS // 16):
        ev = epair[pl.ds(16 * ch, 16)]
        for e in range(E):
            accs[e] = accs[e] + jnp.where(ev == e, 1, 0)
    hist = jnp.zeros((16,), jnp.int32)
    for e in range(E):
        hist = hist + jnp.where(iot == e, _lane_sum(accs[e]), 0)
    histv[...] = hist
    pltpu.sync_copy(histv, shared_hist.at[s])
    plsc.subcore_barrier()

    pltpu.sync_copy(shared_hist, allh)
    tot = jnp.zeros((16,), jnp.int32)
    pre = jnp.zeros((16,), jnp.int32)
    for w in range(NSUB):
        row = allh[w]
        tot = tot + row
        gate = (jnp.int32(w) < s).astype(jnp.int32)
        pre = pre + row * gate
    padded = ((tot + (MM_TILE - 1)) >> 7) << 7
    incl = _lane_prefix(padded)
    base = (incl - padded) + pre

    @pl.when(jnp.logical_and(c == 0, s == 0))
    def _texp():
        ies = [_lane_bcast(incl, e) for e in range(E)]
        for vi in range(NTP // 16):
            startv = (iot + 16 * vi) * MM_TILE
            below = jnp.zeros((16,), jnp.int32)
            for e in range(E):
                below = below + jnp.where(ies[e] <= startv, 1, 0)
            texp_v[pl.ds(16 * vi, 16)] = jnp.minimum(below, E - 1)
        pltpu.sync_copy(texp_v, texp_ref)

    run = base
    for ch in range(PPS // 16):
        ev = epair[pl.ds(16 * ch, 16)]
        sv = jnp.zeros((16,), jnp.int32)
        cvec = jnp.zeros((16,), jnp.int32)
        for e in range(E):
            m = ev == e
            r = _lane_prefix(jnp.where(m, 1, 0))
            sv = jnp.where(m, _lane_bcast(run, e) + (r - 1), sv)
            cvec = cvec + jnp.where(iot == e, _lane_bcast(r, 15), 0)
        run = run + cvec
        slots[pl.ds(16 * ch, 16)] = sv
        toks[pl.ds(16 * ch, 16)] = ((iot + 16 * ch) & (TPS - 1)) + TPS * s

    @pl.when(c == 0)
    def _pos():
        pltpu.sync_copy(slots.at[pl.ds(0, TPS)], pos_ref.at[0, pl.ds(TPS * s, TPS)])
        pltpu.sync_copy(slots.at[pl.ds(TPS, TPS)], pos_ref.at[1, pl.ds(TPS * s, TPS)])

    pltpu.sync_copy(toks, shared_src.at[slots])
    pltpu.sync_copy(wpair, shared_ws.at[slots])
    plsc.subcore_barrier()

    start = SPW * (NSUB * c + s)
    pltpu.sync_copy(shared_ws.at[pl.ds(start, SPW)], ws_v)
    pltpu.sync_copy(ws_v, ws_ref.at[pl.ds(start, SPW)])
    cw = SPW // GCH
    for h in range(GCH):
        pltpu.sync_copy(shared_src.at[pl.ds(start + cw * h, cw)], idxq.at[h])
        for j in range(cw // 16):
            v = idxq[h, pl.ds(16 * j, 16)]
            idxq[h, pl.ds(16 * j, 16)] = jnp.clip(v, 0, T - 1)
    gathers = [
        pltpu.async_copy(hidden_ref.at[idxq.at[h]],
                         rows.at[pl.ds(cw * h, cw)], sem)
        for h in range(GCH)
    ]
    outs = []
    for h in range(GCH):
        gathers[h].wait()
        outs.append(pltpu.async_copy(
            rows.at[pl.ds(cw * h, cw)],
            xs_ref.at[pl.ds(start + cw * h, cw)], sem2))
    for o in outs:
        o.wait()


_dispatch = functools.partial(
    pl.kernel,
    out_type=[
        jax.ShapeDtypeStruct((NS, D), jnp.float32),
        jax.ShapeDtypeStruct((NS,), jnp.float32),
        jax.ShapeDtypeStruct((K, T), jnp.int32),
        jax.ShapeDtypeStruct((NTP,), jnp.int32),
    ],
    mesh=plsc.VectorSubcoreMesh(core_axis_name="c", subcore_axis_name="s"),
    scratch_types=[
        pltpu.VMEM((E, TPS), jnp.float32),
        pltpu.VMEM((PPS,), jnp.int32),
        pltpu.VMEM((PPS,), jnp.float32),
        pltpu.VMEM((16,), jnp.int32),
        pltpu.VMEM((NSUB, 16), jnp.int32),
        pltpu.VMEM((NTP,), jnp.int32),
        pltpu.VMEM((PPS,), jnp.int32),
        pltpu.VMEM((PPS,), jnp.int32),
        pltpu.VMEM((GCH, SPW // GCH), jnp.int32),
        pltpu.VMEM((SPW, D), jnp.float32),
        pltpu.VMEM((SPW,), jnp.float32),
        pltpu.VMEM_SHARED((NSUB, 16), jnp.int32),
        pltpu.VMEM_SHARED((NS,), jnp.int32),
        pltpu.VMEM_SHARED((NS,), jnp.float32),
        pltpu.SemaphoreType.DMA,
        pltpu.SemaphoreType.DMA,
    ],
)(_dispatch_body)


def _mm_body(texp_ref, x_ref, w_ref, b_ref, ws_ref, y_ref):
    y = jax.lax.dot_general(
        x_ref[...].astype(jnp.bfloat16), w_ref[0], (((1,), (0,)), ((), ())),
        preferred_element_type=jnp.float32,
    ) + b_ref[0]
    y_ref[...] = y * ws_ref[0, 0][:, None]


def _combine_body(y_ref, pos_ref, out_ref, p0, p1, y0, y1, sem0, sem1):
    wid = lax.axis_index("s") * NC + lax.axis_index("c")
    tw = T // (NC * NSUB)
    base = tw * wid
    pltpu.sync_copy(pos_ref.at[0, pl.ds(base, tw)], p0)
    pltpu.sync_copy(pos_ref.at[1, pl.ds(base, tw)], p1)
    cp0 = pltpu.async_copy(y_ref.at[p0], y0, sem0)
    cp1 = pltpu.async_copy(y_ref.at[p1], y1, sem1)
    cp0.wait()
    cp1.wait()

    def body(t, carry):
        for v in range(D // 16):
            sl = pl.ds(16 * v, 16)
            y0[t, sl] = y0[t, sl] + y1[t, sl]
        return carry

    lax.fori_loop(0, tw, body, 0)
    pltpu.sync_copy(y0, out_ref.at[pl.ds(base, tw)])


_combine = functools.partial(
    pl.kernel,
    out_type=jax.ShapeDtypeStruct((T, D), jnp.float32),
    mesh=plsc.VectorSubcoreMesh(core_axis_name="c", subcore_axis_name="s"),
    scratch_types=[
        pltpu.VMEM((T // (NC * NSUB),), jnp.int32),
        pltpu.VMEM((T // (NC * NSUB),), jnp.int32),
        pltpu.VMEM((T // (NC * NSUB), D), jnp.float32),
        pltpu.VMEM((T // (NC * NSUB), D), jnp.float32),
        pltpu.SemaphoreType.DMA,
        pltpu.SemaphoreType.DMA,
    ],
)(_combine_body)


def kernel(hidden, gate_logits, W_experts, b_experts):
    xs, ws, pos, texp = _dispatch(gate_logits.T, hidden)
    y = pl.pallas_call(
        _mm_body,
        grid_spec=pltpu.PrefetchScalarGridSpec(
            num_scalar_prefetch=1,
            grid=(NT,),
            in_specs=[
                pl.BlockSpec((MM_TILE, D), lambda i, tx: (i, 0)),
                pl.BlockSpec((1, D, D), lambda i, tx: (tx[i], 0, 0)),
                pl.BlockSpec((1, 1, D), lambda i, tx: (tx[i], 0, 0)),
                pl.BlockSpec((1, 1, MM_TILE), lambda i, tx: (i, 0, 0)),
            ],
            out_specs=pl.BlockSpec((MM_TILE, D), lambda i, tx: (i, 0)),
        ),
        out_shape=jax.ShapeDtypeStruct((NS, D), jnp.float32),
    )(texp, xs, W_experts.astype(jnp.bfloat16), b_experts.reshape(E, 1, D),
      ws.reshape(NT, 1, MM_TILE))
    return _combine(y, pos)


# speedup vs baseline: 4.1345x; 4.0714x over previous
import jax
import jax.numpy as jnp
from jax.experimental import pallas as pl
from jax.experimental.pallas import tpu as pltpu

NUM_EXPERTS = 8
D_MODEL = 768
T_TILE = 256


def _moe_body(logits_ref, x_ref, w_ref, b_ref, out_ref, wb_ref):
    @pl.when(pl.program_id(0) == 0)
    def _cast_w():
        wb_ref[...] = w_ref[...].astype(jnp.bfloat16)

    logits = logits_ref[...]
    x = x_ref[...]
    m1 = jnp.max(logits, axis=-1, keepdims=True)
    i1 = jnp.argmax(logits, axis=-1)[:, None]
    masked = jnp.where(jax.lax.broadcasted_iota(jnp.int32, logits.shape, 1) == i1,
                       jnp.full_like(logits, -jnp.inf), logits)
    m2 = jnp.max(masked, axis=-1, keepdims=True)
    i2 = jnp.argmax(masked, axis=-1)[:, None]
    e2 = jnp.exp(m2 - m1)
    w1 = 1.0 / (1.0 + e2)
    w2 = e2 / (1.0 + e2)
    acc = jnp.zeros_like(x)
    xb = x.astype(jnp.bfloat16)
    for e in range(NUM_EXPERTS):
        ce = jnp.where(i1 == e, w1, jnp.where(i2 == e, w2, 0.0))
        y = jax.lax.dot_general(
            xb, wb_ref[e], (((1,), (0,)), ((), ())),
            preferred_element_type=jnp.float32,
        ) + b_ref[e][None, :]
        acc = acc + ce * y
    out_ref[...] = acc


def kernel(hidden, gate_logits, W_experts, b_experts):
    T, D = hidden.shape
    return pl.pallas_call(
        _moe_body,
        grid=(T // T_TILE,),
        in_specs=[
            pl.BlockSpec((T_TILE, NUM_EXPERTS), lambda i: (i, 0)),
            pl.BlockSpec((T_TILE, D), lambda i: (i, 0)),
            pl.BlockSpec((NUM_EXPERTS, D, D), lambda i: (0, 0, 0)),
            pl.BlockSpec((NUM_EXPERTS, D), lambda i: (0, 0)),
        ],
        out_specs=pl.BlockSpec((T_TILE, D), lambda i: (i, 0)),
        out_shape=jax.ShapeDtypeStruct((T, D), jnp.float32),
        scratch_shapes=[pltpu.VMEM((NUM_EXPERTS, D, D), jnp.bfloat16)],
    )(gate_logits, hidden, W_experts, b_experts)


# speedup vs baseline: 4.2215x; 1.0210x over previous
import jax
import jax.numpy as jnp
from jax.experimental import pallas as pl
from jax.experimental.pallas import tpu as pltpu

NUM_EXPERTS = 8
D_MODEL = 768
T_TILE = 512


def _moe_body(logits_ref, x_ref, w_ref, b_ref, out_ref, wb_ref):
    @pl.when(pl.program_id(0) == 0)
    def _cast_w():
        wb_ref[...] = w_ref[...].astype(jnp.bfloat16)

    logits = logits_ref[...]
    x = x_ref[...]
    m1 = jnp.max(logits, axis=-1, keepdims=True)
    i1 = jnp.argmax(logits, axis=-1)[:, None]
    masked = jnp.where(jax.lax.broadcasted_iota(jnp.int32, logits.shape, 1) == i1,
                       jnp.full_like(logits, -jnp.inf), logits)
    m2 = jnp.max(masked, axis=-1, keepdims=True)
    i2 = jnp.argmax(masked, axis=-1)[:, None]
    e2 = jnp.exp(m2 - m1)
    w1 = 1.0 / (1.0 + e2)
    w2 = e2 / (1.0 + e2)
    acc = jnp.zeros_like(x)
    xb = x.astype(jnp.bfloat16)
    for e in range(NUM_EXPERTS):
        ce = jnp.where(i1 == e, w1, jnp.where(i2 == e, w2, 0.0))
        y = jax.lax.dot_general(
            xb, wb_ref[e], (((1,), (0,)), ((), ())),
            preferred_element_type=jnp.float32,
        ) + b_ref[e][None, :]
        acc = acc + ce * y
    out_ref[...] = acc


def kernel(hidden, gate_logits, W_experts, b_experts):
    T, D = hidden.shape
    return pl.pallas_call(
        _moe_body,
        grid=(T // T_TILE,),
        in_specs=[
            pl.BlockSpec((T_TILE, NUM_EXPERTS), lambda i: (i, 0)),
            pl.BlockSpec((T_TILE, D), lambda i: (i, 0)),
            pl.BlockSpec((NUM_EXPERTS, D, D), lambda i: (0, 0, 0)),
            pl.BlockSpec((NUM_EXPERTS, D), lambda i: (0, 0)),
        ],
        out_specs=pl.BlockSpec((T_TILE, D), lambda i: (i, 0)),
        out_shape=jax.ShapeDtypeStruct((T, D), jnp.float32),
        scratch_shapes=[pltpu.VMEM((NUM_EXPERTS, D, D), jnp.bfloat16)],
    )(gate_logits, hidden, W_experts, b_experts)
